# Initial kernel scaffold; baseline (speedup 1.0000x reference)
import jax
import jax.numpy as jnp
from jax.experimental import pallas as pl


def kernel(h, e, edge_index, W_emb, b_emb, Ws, bs, gammas, betas, Wm0, bm0, Wm1, bm1, Wm2, bm2):
    raise NotImplementedError("write your pallas kernel here")


# speedup vs baseline: 4.7235x; 4.7235x over previous
import functools

import jax
import jax.numpy as jnp
from jax import lax
from jax.experimental import pallas as pl
from jax.experimental.pallas import tpu as pltpu
from jax.experimental.pallas import tpu_sc as plsc

N = 10000
E = 320000
D = 128
NLAYERS = 4

NC = 2
NS = 16
EW = E // (NC * NS)
K = 80
NCHUNK = EW // K
NPAD = 10240
RPT = NPAD // NS

@functools.cache
def _sc_mesh():
    return plsc.VectorSubcoreMesh(core_axis_name="c", subcore_axis_name="s")


def _fill_f32(ref, nrows, ncols, value):
    vec = jnp.full((16,), value, jnp.float32)

    def st(i, _):
        r = i // (ncols // 16)
        col = (i % (ncols // 16)) * 16
        ref[r, pl.ds(col, 16)] = vec
        return ()

    lax.fori_loop(0, nrows * (ncols // 16), st, ())


def _agg_body(x_hbm, src_hbm, dst_hbm, out_hbm, idx_s, idx_d, rows, acc, sem):
    c = lax.axis_index("c")
    s = lax.axis_index("s")

    _fill_f32(rows, K, D, 0.0)
    for j in range(RPT // K):
        pltpu.sync_copy(rows, acc.at[pl.ds(s * RPT + j * K, K)])
    plsc.subcore_barrier()

    ebase = (c * NS + s) * EW

    def step(j, _):
        off = pl.multiple_of(ebase + j * K, 8)
        pltpu.sync_copy(src_hbm.at[pl.ds(off, K)], idx_s)
        pltpu.async_copy(x_hbm.at[idx_s], rows, sem).wait()
        pltpu.sync_copy(dst_hbm.at[pl.ds(off, K)], idx_d)
        pltpu.sync_copy(rows, acc.at[idx_d], add=True)
        return ()

    lax.fori_loop(0, NCHUNK, step, ())
    plsc.subcore_barrier()
    pltpu.sync_copy(acc.at[pl.ds(s * RPT, RPT)], out_hbm.at[c, pl.ds(s * RPT, RPT)])


@functools.cache
def _agg_kernel():
    return pl.kernel(
        _agg_body,
        out_type=jax.ShapeDtypeStruct((NC, NPAD, D), jnp.float32),
        mesh=_sc_mesh(),
        scratch_types=[
            pltpu.VMEM((K,), jnp.int32),
            pltpu.VMEM((K,), jnp.int32),
            pltpu.VMEM((K, D), jnp.float32),
            pltpu.VMEM_SHARED((NPAD, D), jnp.float32),
            pltpu.SemaphoreType.DMA,
        ],
    )


def _agg_call(x, src, dst):
    return _agg_kernel()(x, src, dst)


def _deg_body(dst_hbm, out_hbm, idx_d, rows, acc):
    c = lax.axis_index("c")
    s = lax.axis_index("s")

    _fill_f32(rows, K, D, 0.0)
    for j in range(RPT // K):
        pltpu.sync_copy(rows, acc.at[pl.ds(s * RPT + j * K, K)])
    _fill_f32(rows, K, D, 1.0)
    plsc.subcore_barrier()

    ebase = (c * NS + s) * EW

    def step(j, _):
        off = pl.multiple_of(ebase + j * K, 8)
        pltpu.sync_copy(dst_hbm.at[pl.ds(off, K)], idx_d)
        pltpu.sync_copy(rows, acc.at[idx_d], add=True)
        return ()

    lax.fori_loop(0, NCHUNK, step, ())
    plsc.subcore_barrier()
    pltpu.sync_copy(acc.at[pl.ds(s * RPT, RPT)], out_hbm.at[c, pl.ds(s * RPT, RPT)])


@functools.cache
def _deg_kernel():
    return pl.kernel(
        _deg_body,
        out_type=jax.ShapeDtypeStruct((NC, NPAD, D), jnp.float32),
        mesh=_sc_mesh(),
        scratch_types=[
            pltpu.VMEM((K,), jnp.int32),
            pltpu.VMEM((K, D), jnp.float32),
            pltpu.VMEM_SHARED((NPAD, D), jnp.float32),
        ],
    )


def _deg_call(dst):
    return _deg_kernel()(dst)


def _emb_body(h_ref, W_ref, b_ref, dp_ref, x_ref, inv_ref):
    x_ref[...] = (
        jnp.dot(h_ref[...], W_ref[...], preferred_element_type=jnp.float32)
        + b_ref[...]
    )
    dp = dp_ref[...]
    deg = dp[0, :N, 0:1] + dp[1, :N, 0:1]
    inv_ref[...] = 1.0 / jnp.maximum(deg, 1.0)


def _emb_call(h, W, b, deg_parts):
    return pl.pallas_call(
        _emb_body,
        out_shape=[
            jax.ShapeDtypeStruct((N, D), jnp.float32),
            jax.ShapeDtypeStruct((N, 1), jnp.float32),
        ],
    )(h, W, b, deg_parts)


def _layer_body(x_ref, p_ref, inv_ref, Wx_ref, Wc_ref, b_ref, g_ref, be_ref, o_ref):
    x = x_ref[...]
    cagg = (p_ref[0, :N] + p_ref[1, :N]) * inv_ref[...]
    bundle = (
        jnp.dot(x, Wx_ref[...], preferred_element_type=jnp.float32)
        + jnp.dot(cagg, Wc_ref[...], preferred_element_type=jnp.float32)
        + b_ref[...]
    )
    nrm = jnp.sqrt(jnp.sum(bundle * bundle, axis=1, keepdims=True))
    bundle = bundle / jnp.maximum(nrm, 1e-12)
    bundle = jnp.maximum(bundle, 0.0)
    mean = jnp.mean(bundle, axis=0, keepdims=True)
    ctr = bundle - mean
    var = jnp.mean(ctr * ctr, axis=0, keepdims=True)
    bundle = ctr * (g_ref[...] / jnp.sqrt(var + 1e-5)) + be_ref[...]
    o_ref[...] = x + bundle


def _layer_call(x, parts, inv, Wx, Wc, b, g, be):
    return pl.pallas_call(
        _layer_body,
        out_shape=jax.ShapeDtypeStruct((N, D), jnp.float32),
    )(x, parts, inv, Wx, Wc, b, g, be)


def _readout_body(x_ref, W0_ref, b0_ref, W1_ref, b1_ref, W2_ref, b2_ref, y_ref):
    y = jnp.maximum(
        jnp.dot(x_ref[...], W0_ref[...], preferred_element_type=jnp.float32)
        + b0_ref[...], 0.0)
    y = jnp.maximum(
        jnp.dot(y, W1_ref[...], preferred_element_type=jnp.float32)
        + b1_ref[...], 0.0)
    y_ref[...] = (
        jnp.dot(y, W2_ref[...], preferred_element_type=jnp.float32)
        + b2_ref[...])


def _readout_call(x, W0, b0, W1, b1, W2, b2):
    return pl.pallas_call(
        _readout_body,
        out_shape=jax.ShapeDtypeStruct((N, 40), jnp.float32),
    )(x, W0, b0, W1, b1, W2, b2)


def kernel(h, e, edge_index, W_emb, b_emb, Ws, bs, gammas, betas,
           Wm0, bm0, Wm1, bm1, Wm2, bm2):
    src = edge_index[0]
    dst = edge_index[1]
    deg_parts = _deg_call(dst)
    x, inv = _emb_call(h, W_emb, b_emb.reshape(1, D), deg_parts)
    for l in range(NLAYERS):
        parts = _agg_call(x, src, dst)
        x = _layer_call(x, parts, inv, Ws[l, :D], Ws[l, D:],
                        bs[l].reshape(1, D), gammas[l].reshape(1, D),
                        betas[l].reshape(1, D))
    y = _readout_call(x, Wm0, bm0.reshape(1, -1), Wm1, bm1.reshape(1, -1),
                      Wm2, bm2.reshape(1, -1))
    return y


# speedup vs baseline: 10.4249x; 2.2070x over previous
import functools

import jax
import jax.numpy as jnp
from jax import lax
from jax.experimental import pallas as pl
from jax.experimental.pallas import tpu as pltpu
from jax.experimental.pallas import tpu_sc as plsc

N = 10000
E = 320000
D = 128
NLAYERS = 4

NC = 2
NS = 16
EW = E // (NC * NS)
K = 80
NCHUNK = EW // K
NPAD = 10240
RPT = NPAD // NS

@functools.cache
def _sc_mesh():
    return plsc.VectorSubcoreMesh(core_axis_name="c", subcore_axis_name="s")


def _fill_f32(ref, nrows, ncols, value):
    vec = jnp.full((16,), value, jnp.float32)

    def st(i, _):
        r = i // (ncols // 16)
        col = (i % (ncols // 16)) * 16
        ref[r, pl.ds(col, 16)] = vec
        return ()

    lax.fori_loop(0, nrows * (ncols // 16), st, ())


def _agg_body(x_hbm, src_hbm, dst_hbm, out_hbm,
              sidx, didx, rows0, rows1, acc, sem0, sem1):
    c = lax.axis_index("c")
    s = lax.axis_index("s")
    w = c * NS + s

    pltpu.sync_copy(src_hbm.at[w], sidx)
    pltpu.sync_copy(dst_hbm.at[w], didx)

    _fill_f32(rows0, K, D, 0.0)
    for j in range(RPT // K):
        pltpu.sync_copy(rows0, acc.at[pl.ds(s * RPT + j * K, K)])
    plsc.subcore_barrier()

    def gather(j, rows, sem):
        pltpu.async_copy(x_hbm.at[sidx.at[pl.ds(j * K, K)]], rows, sem)

    def drain(rows, sem):
        pltpu.make_async_copy(x_hbm.at[pl.ds(0, K)], rows, sem).wait()

    def scat(j, rows):
        pltpu.sync_copy(rows, acc.at[didx.at[j]], add=True)

    gather(0, rows0, sem0)
    gather(1, rows1, sem1)

    def step(i, _):
        j = 2 * i
        drain(rows0, sem0)
        scat(j, rows0)
        gather(j + 2, rows0, sem0)
        drain(rows1, sem1)
        scat(j + 1, rows1)
        gather(j + 3, rows1, sem1)
        return ()

    lax.fori_loop(0, (NCHUNK - 3) // 2, step, ())
    drain(rows0, sem0)
    scat(NCHUNK - 3, rows0)
    gather(NCHUNK - 1, rows0, sem0)
    drain(rows1, sem1)
    scat(NCHUNK - 2, rows1)
    drain(rows0, sem0)
    scat(NCHUNK - 1, rows0)

    plsc.subcore_barrier()
    pltpu.sync_copy(acc.at[pl.ds(s * RPT, RPT)], out_hbm.at[c, pl.ds(s * RPT, RPT)])


@functools.cache
def _agg_kernel():
    return pl.kernel(
        _agg_body,
        out_type=jax.ShapeDtypeStruct((NC, NPAD, D), jnp.float32),
        mesh=_sc_mesh(),
        scratch_types=[
            pltpu.VMEM((EW,), jnp.int32),
            pltpu.VMEM((NCHUNK, K), jnp.int32),
            pltpu.VMEM((K, D), jnp.float32),
            pltpu.VMEM((K, D), jnp.float32),
            pltpu.VMEM_SHARED((NPAD, D), jnp.float32),
            pltpu.SemaphoreType.DMA,
            pltpu.SemaphoreType.DMA,
        ],
    )


def _agg_call(x, src3, dst3):
    return _agg_kernel()(x, src3, dst3)


def _deg_body(dst_hbm, out_hbm, didx, rows, acc):
    c = lax.axis_index("c")
    s = lax.axis_index("s")
    w = c * NS + s

    pltpu.sync_copy(dst_hbm.at[w], didx)

    _fill_f32(rows, K, D, 0.0)
    for j in range(RPT // K):
        pltpu.sync_copy(rows, acc.at[pl.ds(s * RPT + j * K, K)])
    _fill_f32(rows, K, D, 1.0)
    plsc.subcore_barrier()

    def step(j, _):
        pltpu.sync_copy(rows, acc.at[didx.at[j]], add=True)
        return ()

    lax.fori_loop(0, NCHUNK, step, ())
    plsc.subcore_barrier()
    pltpu.sync_copy(acc.at[pl.ds(s * RPT, RPT)], out_hbm.at[c, pl.ds(s * RPT, RPT)])


@functools.cache
def _deg_kernel():
    return pl.kernel(
        _deg_body,
        out_type=jax.ShapeDtypeStruct((NC, NPAD, D), jnp.float32),
        mesh=_sc_mesh(),
        scratch_types=[
            pltpu.VMEM((NCHUNK, K), jnp.int32),
            pltpu.VMEM((K, D), jnp.float32),
            pltpu.VMEM_SHARED((NPAD, D), jnp.float32),
        ],
    )


def _deg_call(dst):
    return _deg_kernel()(dst)


def _emb_body(h_ref, W_ref, b_ref, dp_ref, x_ref, inv_ref):
    x_ref[...] = (
        jnp.dot(h_ref[...], W_ref[...], preferred_element_type=jnp.float32)
        + b_ref[...]
    )
    dp = dp_ref[...]
    deg = dp[0, :N, 0:1] + dp[1, :N, 0:1]
    inv_ref[...] = 1.0 / jnp.maximum(deg, 1.0)


def _emb_call(h, W, b, deg_parts):
    return pl.pallas_call(
        _emb_body,
        out_shape=[
            jax.ShapeDtypeStruct((N, D), jnp.float32),
            jax.ShapeDtypeStruct((N, 1), jnp.float32),
        ],
    )(h, W, b, deg_parts)


def _layer_body(x_ref, p_ref, inv_ref, Wx_ref, Wc_ref, b_ref, g_ref, be_ref, o_ref):
    x = x_ref[...]
    cagg = (p_ref[0, :N] + p_ref[1, :N]) * inv_ref[...]
    bundle = (
        jnp.dot(x, Wx_ref[...], preferred_element_type=jnp.float32)
        + jnp.dot(cagg, Wc_ref[...], preferred_element_type=jnp.float32)
        + b_ref[...]
    )
    nrm = jnp.sqrt(jnp.sum(bundle * bundle, axis=1, keepdims=True))
    bundle = bundle / jnp.maximum(nrm, 1e-12)
    bundle = jnp.maximum(bundle, 0.0)
    mean = jnp.mean(bundle, axis=0, keepdims=True)
    ctr = bundle - mean
    var = jnp.mean(ctr * ctr, axis=0, keepdims=True)
    bundle = ctr * (g_ref[...] / jnp.sqrt(var + 1e-5)) + be_ref[...]
    o_ref[...] = x + bundle


def _layer_call(x, parts, inv, Wx, Wc, b, g, be):
    return pl.pallas_call(
        _layer_body,
        out_shape=jax.ShapeDtypeStruct((N, D), jnp.float32),
    )(x, parts, inv, Wx, Wc, b, g, be)


def _readout_body(x_ref, W0_ref, b0_ref, W1_ref, b1_ref, W2_ref, b2_ref, y_ref):
    y = jnp.maximum(
        jnp.dot(x_ref[...], W0_ref[...], preferred_element_type=jnp.float32)
        + b0_ref[...], 0.0)
    y = jnp.maximum(
        jnp.dot(y, W1_ref[...], preferred_element_type=jnp.float32)
        + b1_ref[...], 0.0)
    y_ref[...] = (
        jnp.dot(y, W2_ref[...], preferred_element_type=jnp.float32)
        + b2_ref[...])


def _readout_call(x, W0, b0, W1, b1, W2, b2):
    return pl.pallas_call(
        _readout_body,
        out_shape=jax.ShapeDtypeStruct((N, 40), jnp.float32),
    )(x, W0, b0, W1, b1, W2, b2)


def kernel(h, e, edge_index, W_emb, b_emb, Ws, bs, gammas, betas,
           Wm0, bm0, Wm1, bm1, Wm2, bm2):
    src = edge_index[0].reshape(NC * NS, EW)
    dst = edge_index[1].reshape(NC * NS, NCHUNK, K)
    deg_parts = _deg_call(dst)
    x, inv = _emb_call(h, W_emb, b_emb.reshape(1, D), deg_parts)
    for l in range(NLAYERS):
        parts = _agg_call(x, src, dst)
        x = _layer_call(x, parts, inv, Ws[l, :D], Ws[l, D:],
                        bs[l].reshape(1, D), gammas[l].reshape(1, D),
                        betas[l].reshape(1, D))
    y = _readout_call(x, Wm0, bm0.reshape(1, -1), Wm1, bm1.reshape(1, -1),
                      Wm2, bm2.reshape(1, -1))
    return y
